# baseline (speedup 1.0000x reference)
import functools

import jax
import jax.numpy as jnp
from jax import lax
from jax.experimental import pallas as pl
from jax.experimental.pallas import tpu as pltpu
from jax.experimental.pallas import tpu_sc as plsc


def kernel(embed_user, embed_item):
    n, d = embed_user.shape
    info = plsc.get_sparse_core_info()
    num_cores, num_subcores = info.num_cores, info.num_subcores
    nw = num_cores * num_subcores
    rows = (n // nw) // 8 * 8
    rem = n - nw * rows

    mesh = plsc.VectorSubcoreMesh(core_axis_name="c", subcore_axis_name="s")

    @functools.partial(
        pl.kernel,
        mesh=mesh,
        out_type=jax.ShapeDtypeStruct((2, n, d), embed_user.dtype),
        scratch_types=[pltpu.SemaphoreType.DMA, pltpu.SemaphoreType.DMA],
    )
    def copy_tables(user_hbm, item_hbm, out_hbm, sem_u, sem_i):
        wid = lax.axis_index("s") * num_cores + lax.axis_index("c")
        base = wid * rows
        cu = pltpu.make_async_copy(
            user_hbm.at[pl.ds(base, rows)],
            out_hbm.at[0, pl.ds(base, rows)],
            sem_u,
        )
        ci = pltpu.make_async_copy(
            item_hbm.at[pl.ds(base, rows)],
            out_hbm.at[1, pl.ds(base, rows)],
            sem_i,
        )
        cu.start()
        ci.start()
        if rem:
            tail = nw * rows

            @pl.when(wid == 0)
            def _():
                tu = pltpu.make_async_copy(
                    user_hbm.at[pl.ds(tail, rem)],
                    out_hbm.at[0, pl.ds(tail, rem)],
                    sem_u,
                )
                ti = pltpu.make_async_copy(
                    item_hbm.at[pl.ds(tail, rem)],
                    out_hbm.at[1, pl.ds(tail, rem)],
                    sem_i,
                )
                tu.start()
                ti.start()
                tu.wait()
                ti.wait()

        cu.wait()
        ci.wait()

    return copy_tables(embed_user, embed_item)


# speedup vs baseline: 1.0051x; 1.0051x over previous
import jax
import jax.numpy as jnp
from jax.experimental import pallas as pl
from jax.experimental.pallas import tpu as pltpu


def _copy_body(user_hbm, item_hbm, out_hbm, sem_u, sem_i):
    cu = pltpu.make_async_copy(user_hbm, out_hbm.at[0], sem_u)
    ci = pltpu.make_async_copy(item_hbm, out_hbm.at[1], sem_i)
    cu.start()
    ci.start()
    cu.wait()
    ci.wait()


def kernel(embed_user, embed_item):
    n, d = embed_user.shape
    return pl.pallas_call(
        _copy_body,
        out_shape=jax.ShapeDtypeStruct((2, n, d), embed_user.dtype),
        in_specs=[
            pl.BlockSpec(memory_space=pltpu.MemorySpace.HBM),
            pl.BlockSpec(memory_space=pltpu.MemorySpace.HBM),
        ],
        out_specs=pl.BlockSpec(memory_space=pltpu.MemorySpace.HBM),
        scratch_shapes=[pltpu.SemaphoreType.DMA, pltpu.SemaphoreType.DMA],
    )(embed_user, embed_item)


# speedup vs baseline: 48.0567x; 47.8107x over previous
import jax
import jax.numpy as jnp
from jax.experimental import pallas as pl
from jax.experimental.pallas import tpu as pltpu

_BLOCK_ROWS = 4000


def _copy_body(user_ref, item_ref, out_ref):
    out_ref[0] = user_ref[...]
    out_ref[1] = item_ref[...]


def kernel(embed_user, embed_item):
    n, d = embed_user.shape
    bn = _BLOCK_ROWS if n % _BLOCK_ROWS == 0 else n
    grid = (n // bn,)
    return pl.pallas_call(
        _copy_body,
        grid=grid,
        in_specs=[
            pl.BlockSpec((bn, d), lambda j: (j, 0)),
            pl.BlockSpec((bn, d), lambda j: (j, 0)),
        ],
        out_specs=pl.BlockSpec((2, bn, d), lambda j: (0, j, 0)),
        out_shape=jax.ShapeDtypeStruct((2, n, d), embed_user.dtype),
    )(embed_user, embed_item)
